# baseline (speedup 1.0000x reference)
import functools

import jax
import jax.numpy as jnp
from jax.experimental import pallas as pl
from jax.experimental.pallas import tpu as pltpu


def _sage_layer_body(adj_ref, xa_ref, xs_ref, ws_ref, wn_ref, b_ref, out_ref,
                     *, feat, apply_relu):
    prod = jnp.dot(adj_ref[...], xa_ref[...],
                   preferred_element_type=jnp.float32,
                   precision=jax.lax.Precision.DEFAULT)
    s = prod[:, :feat]
    deg = jnp.clip(prod[:, feat:feat + 1], 1e-6, None)
    neigh = s / deg
    h = (jnp.dot(xs_ref[...], ws_ref[...], preferred_element_type=jnp.float32,
                 precision=jax.lax.Precision.HIGHEST)
         + jnp.dot(neigh, wn_ref[...], preferred_element_type=jnp.float32,
                   precision=jax.lax.Precision.HIGHEST)
         + b_ref[...])
    if apply_relu:
        h = jnp.maximum(h, 0.0)
    out_ref[...] = h


def _pick_bm(n):
    for c in (400, 256, 200, 128, 80, 64, 40, 32, 16, 8):
        if n % c == 0:
            return c
    return n


def _sage_layer(adj, x_nbr, x_self, w, b, apply_relu):
    nrows, ncols = adj.shape
    feat = x_nbr.shape[1]
    bm = _pick_bm(nrows)
    xa = jnp.concatenate(
        [x_nbr, jnp.ones((ncols, 1), jnp.float32)], axis=1)
    ws = w[:feat]
    wn = w[feat:]
    b2 = b.reshape(1, feat)
    body = functools.partial(_sage_layer_body, feat=feat, apply_relu=apply_relu)
    return pl.pallas_call(
        body,
        grid=(nrows // bm,),
        in_specs=[
            pl.BlockSpec((bm, ncols), lambda i: (i, 0)),
            pl.BlockSpec((ncols, feat + 1), lambda i: (0, 0)),
            pl.BlockSpec((bm, feat), lambda i: (i, 0)),
            pl.BlockSpec((feat, feat), lambda i: (0, 0)),
            pl.BlockSpec((feat, feat), lambda i: (0, 0)),
            pl.BlockSpec((1, feat), lambda i: (0, 0)),
        ],
        out_specs=pl.BlockSpec((bm, feat), lambda i: (i, 0)),
        out_shape=jax.ShapeDtypeStruct((nrows, feat), jnp.float32),
        compiler_params=pltpu.CompilerParams(
            dimension_semantics=("arbitrary",),
        ),
    )(adj, xa, x_self, ws, wn, b2)


def kernel(fts, adj, W1, b1, W2, b2):
    h = _sage_layer(adj, fts, fts, W1, b1, apply_relu=True)
    return _sage_layer(adj, h, h, W2, b2, apply_relu=False)


# speedup vs baseline: 1.0471x; 1.0471x over previous
import functools

import jax
import jax.numpy as jnp
from jax.experimental import pallas as pl
from jax.experimental.pallas import tpu as pltpu


def _fused_body(adj_ref, xa0_ref, xs_ref, ws_ref, wn_ref, b_ref, out_ref,
                h_s, *, feat, bm):
    l = pl.program_id(0)
    i = pl.program_id(1)
    a = adj_ref[...].astype(jnp.bfloat16)
    base = pl.multiple_of(i * bm, bm)

    def _epilogue(prod, xs):
        s = prod[:, :feat]
        deg = jnp.clip(prod[:, feat:feat + 1], 1e-6, None)
        neigh = s / deg
        return (jnp.dot(xs, ws_ref[0], preferred_element_type=jnp.float32,
                        precision=jax.lax.Precision.HIGHEST)
                + jnp.dot(neigh, wn_ref[0],
                          preferred_element_type=jnp.float32,
                          precision=jax.lax.Precision.HIGHEST)
                + b_ref[0])

    @pl.when(l == 0)
    def _layer1():
        prod = jnp.dot(a, xa0_ref[...], preferred_element_type=jnp.float32)
        h = jnp.maximum(_epilogue(prod, xs_ref[...]), 0.0)
        h_s[pl.ds(base, bm), :feat] = h.astype(jnp.bfloat16)
        h_s[pl.ds(base, bm), feat:feat + 1] = jnp.ones((bm, 1), jnp.bfloat16)
        out_ref[...] = h

    @pl.when(l == 1)
    def _layer2():
        prod = jnp.dot(a, h_s[...], preferred_element_type=jnp.float32)
        xs2 = h_s[pl.ds(base, bm), :feat].astype(jnp.float32)
        out_ref[...] = _epilogue(prod, xs2)


def _pick_bm(n):
    for c in (400, 256, 200, 128, 80, 64, 40, 32, 16, 8):
        if n % c == 0:
            return c
    return n


def kernel(fts, adj, W1, b1, W2, b2):
    n, feat = fts.shape
    bm = _pick_bm(n)
    xa0 = jnp.concatenate(
        [fts.astype(jnp.bfloat16), jnp.ones((n, 1), jnp.bfloat16)], axis=1)
    ws = jnp.stack([W1[:feat], W2[:feat]])
    wn = jnp.stack([W1[feat:], W2[feat:]])
    bb = jnp.stack([b1.reshape(1, feat), b2.reshape(1, feat)])
    body = functools.partial(_fused_body, feat=feat, bm=bm)
    return pl.pallas_call(
        body,
        grid=(2, n // bm),
        in_specs=[
            pl.BlockSpec((bm, n), lambda l, i: (i, 0)),
            pl.BlockSpec((n, feat + 1), lambda l, i: (0, 0)),
            pl.BlockSpec((bm, feat), lambda l, i: (i, 0)),
            pl.BlockSpec((1, feat, feat), lambda l, i: (l, 0, 0)),
            pl.BlockSpec((1, feat, feat), lambda l, i: (l, 0, 0)),
            pl.BlockSpec((1, 1, feat), lambda l, i: (l, 0, 0)),
        ],
        out_specs=pl.BlockSpec((bm, feat), lambda l, i: (i, 0)),
        out_shape=jax.ShapeDtypeStruct((n, feat), jnp.float32),
        scratch_shapes=[pltpu.VMEM((n, feat + 1), jnp.bfloat16)],
        compiler_params=pltpu.CompilerParams(
            dimension_semantics=("arbitrary", "arbitrary"),
        ),
    )(adj, xa0, fts, ws, wn, bb)
